# baseline (speedup 1.0000x reference)
import functools

import jax
import jax.numpy as jnp
from jax import lax
from jax.experimental import pallas as pl
from jax.experimental.pallas import tpu as pltpu
from jax.experimental.pallas import tpu_sc as plsc

ENT = 1000000
REL = 1000
D = 32
B = 16384
LMBDA = 0.0001

NC = 2
NS = 16
NW = NC * NS
BPW = B // NW
CHUNK = 128
NCH = BPW // CHUNK

_mesh = plsc.VectorSubcoreMesh(core_axis_name="c", subcore_axis_name="s")


@functools.partial(
    pl.kernel,
    mesh=_mesh,
    compiler_params=pltpu.CompilerParams(
        needs_layout_passes=False, use_tc_tiling_on_sc=False),
    out_type=[
        jax.ShapeDtypeStruct((B,), jnp.float32),
        jax.ShapeDtypeStruct((B,), jnp.float32),
        jax.ShapeDtypeStruct((NW, 16), jnp.float32),
    ],
    scratch_types=[
        pltpu.VMEM((NCH, CHUNK), jnp.int32),
        pltpu.VMEM((NCH, CHUNK), jnp.int32),
        pltpu.VMEM((NCH, CHUNK), jnp.int32),
        pltpu.VMEM((BPW, D), jnp.float32),
        pltpu.VMEM((BPW, D), jnp.float32),
        pltpu.VMEM((BPW, D), jnp.float32),
        pltpu.VMEM((BPW, D), jnp.float32),
        pltpu.VMEM((BPW, D), jnp.float32),
        pltpu.VMEM((BPW, D), jnp.float32),
        pltpu.VMEM((BPW,), jnp.float32),
        pltpu.VMEM((16,), jnp.float32),
        pltpu.SemaphoreType.DMA,
    ],
)
def _sc_score(ph, pt, pr, nh, nt, nr, ent1, ent2, rel1, rel2,
              ps_out, ns_out, sq_out,
              idx_h, idx_t, idx_r, e1h, e2h, e1t, e2t, r1v, r2v,
              score_v, sq_v, sem):
    wid = lax.axis_index("s") * NC + lax.axis_index("c")
    rbase = wid * NCH
    sbase = wid * BPW

    total_sq = jnp.zeros((16,), jnp.float32)
    for hh, tt, rr, out_ref in ((ph, pt, pr, ps_out), (nh, nt, nr, ns_out)):
        pltpu.sync_copy(hh.at[pl.ds(rbase, NCH)], idx_h)
        pltpu.sync_copy(tt.at[pl.ds(rbase, NCH)], idx_t)
        pltpu.sync_copy(rr.at[pl.ds(rbase, NCH)], idx_r)
        copies = []
        for j in range(NCH):
            sl = pl.ds(j * CHUNK, CHUNK)
            copies.append(pltpu.async_copy(ent1.at[idx_h.at[j]], e1h.at[sl], sem))
            copies.append(pltpu.async_copy(ent2.at[idx_h.at[j]], e2h.at[sl], sem))
            copies.append(pltpu.async_copy(ent1.at[idx_t.at[j]], e1t.at[sl], sem))
            copies.append(pltpu.async_copy(ent2.at[idx_t.at[j]], e2t.at[sl], sem))
            copies.append(pltpu.async_copy(rel1.at[idx_r.at[j]], r1v.at[sl], sem))
            copies.append(pltpu.async_copy(rel2.at[idx_r.at[j]], r2v.at[sl], sem))
        for c in copies:
            c.wait()

        lane_ids = lax.iota(jnp.int32, 16)

        def group(g, acc):
            base = g * 16
            svec = jnp.zeros((16,), jnp.float32)
            for k in range(16):
                r = base + k
                s = jnp.zeros((16,), jnp.float32)
                for h0 in (0, 16):
                    a = e1h[r, pl.ds(h0, 16)]
                    b = e2h[r, pl.ds(h0, 16)]
                    c_ = e1t[r, pl.ds(h0, 16)]
                    d_ = e2t[r, pl.ds(h0, 16)]
                    p = r1v[r, pl.ds(h0, 16)]
                    q = r2v[r, pl.ds(h0, 16)]
                    s = s + (a * c_ + b * d_) * p + (a * d_ - b * c_) * q
                    acc = acc + (a * a + b * b) + (c_ * c_ + d_ * d_) + (p * p + q * q)
                svec = jnp.where(lane_ids == k, jnp.sum(s), svec)
            score_v[pl.ds(base, 16)] = svec
            return acc

        total_sq = total_sq + lax.fori_loop(0, BPW // 16, group, jnp.zeros((16,), jnp.float32))
        pltpu.sync_copy(score_v, out_ref.at[pl.ds(sbase, BPW)])

    sq_v[...] = total_sq
    pltpu.sync_copy(sq_v, sq_out.at[wid])


def _tc_finish_body(ps_ref, ns_ref, py_ref, ny_ref, sq_ref, o_ref):
    x = -py_ref[...] * ps_ref[...]
    y = -ny_ref[...] * ns_ref[...]
    sp = jnp.maximum(x, 0.0) + jnp.log1p(jnp.exp(-jnp.abs(x)))
    sn = jnp.maximum(y, 0.0) + jnp.log1p(jnp.exp(-jnp.abs(y)))
    loss = (jnp.sum(sp) + jnp.sum(sn)) / B
    reg = jnp.sum(sq_ref[...]) / (B * D)
    o_ref[0, 0] = loss + LMBDA * reg


_tc_finish = pl.pallas_call(
    _tc_finish_body,
    out_shape=jax.ShapeDtypeStruct((1, 1), jnp.float32),
    out_specs=pl.BlockSpec(memory_space=pltpu.SMEM),
)


def kernel(pos_h, pos_t, pos_r, neg_h, neg_t, neg_r, pos_y, neg_y,
           ent1, ent2, rel1, rel2):
    ph = pos_h.astype(jnp.int32).reshape(NW * NCH, CHUNK)
    pt = pos_t.astype(jnp.int32).reshape(NW * NCH, CHUNK)
    pr = pos_r.astype(jnp.int32).reshape(NW * NCH, CHUNK)
    nh = neg_h.astype(jnp.int32).reshape(NW * NCH, CHUNK)
    nt = neg_t.astype(jnp.int32).reshape(NW * NCH, CHUNK)
    nr = neg_r.astype(jnp.int32).reshape(NW * NCH, CHUNK)
    ps, ns, sq = _sc_score(ph, pt, pr, nh, nt, nr, ent1, ent2, rel1, rel2)
    out = _tc_finish(ps.reshape(128, 128), ns.reshape(128, 128),
                     pos_y.reshape(128, 128), neg_y.reshape(128, 128), sq)
    return out[0, 0]
